# baseline (speedup 1.0000x reference)
import jax
import jax.numpy as jnp
from jax import lax
from jax.experimental import pallas as pl
from jax.experimental.pallas import tpu as pltpu
from jax.experimental.pallas import tpu_sc as plsc

NE = 1024
D = 64
BATCH = 8
SEQ = 1024
COMMIT = 0.25


BS = 4096


def _dist_argmin_body(x_ref, e_ref, idx_ref, loss_ref):
    i = pl.program_id(0)
    x = x_ref[...]
    e = e_ref[...]
    mmT = lax.dot_general(e, x, (((1,), (1,)), ((), ())),
                          preferred_element_type=jnp.float32)
    x2 = jnp.sum(x * x, axis=1, keepdims=True)
    x2r = lax.transpose(x2, (1, 0))
    e2 = jnp.sum(e * e, axis=1, keepdims=True)
    d = x2r + e2 - 2.0 * mmT
    m = jnp.min(d, axis=0, keepdims=True)
    iota = lax.broadcasted_iota(jnp.int32, (NE, 1), 0).astype(jnp.float32)
    fidx = jnp.min(jnp.where(d == m, iota, float(NE)), axis=0)
    idx_ref[0, 0] = fidx.astype(jnp.int32)

    @pl.when(i == 0)
    def _():
        loss_ref[0] = 0.0

    loss_ref[0] += jnp.sum(m)

    @pl.when(i == pl.num_programs(0) - 1)
    def _():
        loss_ref[0] = loss_ref[0] * (COMMIT / (BATCH * SEQ * D))


@jax.jit
def _dist_argmin(x_flat, embedding):
    nblk = (BATCH * SEQ) // BS
    return pl.pallas_call(
        _dist_argmin_body,
        grid=(nblk,),
        in_specs=[
            pl.BlockSpec((BS, D), lambda i: (i, 0)),
            pl.BlockSpec((NE, D), lambda i: (0, 0)),
        ],
        out_specs=[
            pl.BlockSpec((1, 1, BS), lambda i: (i, 0, 0)),
            pl.BlockSpec(memory_space=pltpu.SMEM),
        ],
        out_shape=[
            jax.ShapeDtypeStruct((nblk, 1, BS), jnp.int32),
            jax.ShapeDtypeStruct((1,), jnp.float32),
        ],
        compiler_params=pltpu.CompilerParams(
            dimension_semantics=("arbitrary",)),
    )(x_flat, embedding)


NC = 1
NS = 16
NW = NC * NS
ROWS = BATCH * SEQ
RPW = ROWS // NW
CHUNK = 128


def _sc_gather_body(emb_hbm, idx_hbm, out_hbm, idx_v, rows_v, sem):
    wid = lax.axis_index("s") * NC + lax.axis_index("c")
    base = wid * RPW
    pltpu.sync_copy(idx_hbm.at[pl.ds(wid * (RPW // CHUNK), RPW // CHUNK)],
                    idx_v)
    copies = [
        pltpu.async_copy(emb_hbm.at[idx_v.at[k]],
                         rows_v.at[pl.ds(k * CHUNK, CHUNK)], sem)
        for k in range(RPW // CHUNK)
    ]
    for c in copies:
        c.wait()
    pltpu.sync_copy(rows_v, out_hbm.at[pl.ds(base, RPW)])


@jax.jit
def _sc_gather(embedding, flat_idx):
    f = pl.kernel(
        _sc_gather_body,
        mesh=plsc.VectorSubcoreMesh(core_axis_name="c", subcore_axis_name="s", num_cores=1),
        out_type=jax.ShapeDtypeStruct((ROWS, D), jnp.float32),
        scratch_types=[
            pltpu.VMEM((RPW // CHUNK, CHUNK), jnp.int32),
            pltpu.VMEM((RPW, D), jnp.float32),
            pltpu.SemaphoreType.DMA,
        ],
        compiler_params=pltpu.CompilerParams(use_tc_tiling_on_sc=False),
    )
    return f(embedding, flat_idx.reshape(ROWS // CHUNK, CHUNK))


def kernel(inputs, embedding):
    x_flat = inputs.reshape(ROWS, D)
    idx2, loss = _dist_argmin(x_flat, embedding)
    qst = _sc_gather(embedding, idx2.reshape(ROWS))
    return (qst.reshape(inputs.shape), loss.reshape(()),
            idx2.reshape(BATCH, SEQ))


# speedup vs baseline: 1.0033x; 1.0033x over previous
import jax
import jax.numpy as jnp
from jax import lax
from jax.experimental import pallas as pl
from jax.experimental.pallas import tpu as pltpu
from jax.experimental.pallas import tpu_sc as plsc

NE = 1024
D = 64
BATCH = 8
SEQ = 1024
COMMIT = 0.25


BS = 4096


def _dist_argmin_body(x_ref, e_ref, idx_ref, loss_ref):
    i = pl.program_id(0)
    x = x_ref[...]
    e = e_ref[...]
    mmT = lax.dot_general(e, x, (((1,), (1,)), ((), ())),
                          preferred_element_type=jnp.float32)
    x2 = jnp.sum(x * x, axis=1, keepdims=True)
    x2r = lax.transpose(x2, (1, 0))
    e2 = jnp.sum(e * e, axis=1, keepdims=True)
    d = x2r + e2 - 2.0 * mmT
    m = jnp.min(d, axis=0, keepdims=True)
    iota = lax.broadcasted_iota(jnp.int32, (NE, 1), 0).astype(jnp.float32)
    fidx = jnp.min(jnp.where(d == m, iota, float(NE)), axis=0)
    idx_ref[0, 0] = fidx.astype(jnp.int32)

    @pl.when(i == 0)
    def _():
        loss_ref[0] = 0.0

    loss_ref[0] += jnp.sum(m)

    @pl.when(i == pl.num_programs(0) - 1)
    def _():
        loss_ref[0] = loss_ref[0] * (COMMIT / (BATCH * SEQ * D))


@jax.jit
def _dist_argmin(x_flat, embedding):
    nblk = (BATCH * SEQ) // BS
    return pl.pallas_call(
        _dist_argmin_body,
        grid=(nblk,),
        in_specs=[
            pl.BlockSpec((BS, D), lambda i: (i, 0)),
            pl.BlockSpec((NE, D), lambda i: (0, 0)),
        ],
        out_specs=[
            pl.BlockSpec((1, 1, BS), lambda i: (i, 0, 0)),
            pl.BlockSpec(memory_space=pltpu.SMEM),
        ],
        out_shape=[
            jax.ShapeDtypeStruct((nblk, 1, BS), jnp.int32),
            jax.ShapeDtypeStruct((1,), jnp.float32),
        ],
        compiler_params=pltpu.CompilerParams(
            dimension_semantics=("arbitrary",),
            vmem_limit_bytes=100 * 1024 * 1024),
    )(x_flat, embedding)


NC = 1
NS = 16
NW = NC * NS
ROWS = BATCH * SEQ
RPW = ROWS // NW
CHUNK = 128


def _sc_gather_body(emb_hbm, idx_hbm, out_hbm, idx_v, rows_v, sem):
    wid = lax.axis_index("s") * NC + lax.axis_index("c")
    base = wid * RPW
    pltpu.sync_copy(idx_hbm.at[pl.ds(wid * (RPW // CHUNK), RPW // CHUNK)],
                    idx_v)
    copies = [
        pltpu.async_copy(emb_hbm.at[idx_v.at[k]],
                         rows_v.at[pl.ds(k * CHUNK, CHUNK)], sem)
        for k in range(RPW // CHUNK)
    ]
    for c in copies:
        c.wait()
    pltpu.sync_copy(rows_v, out_hbm.at[pl.ds(base, RPW)])


@jax.jit
def _sc_gather(embedding, flat_idx):
    f = pl.kernel(
        _sc_gather_body,
        mesh=plsc.VectorSubcoreMesh(core_axis_name="c", subcore_axis_name="s", num_cores=1),
        out_type=jax.ShapeDtypeStruct((ROWS, D), jnp.float32),
        scratch_types=[
            pltpu.VMEM((RPW // CHUNK, CHUNK), jnp.int32),
            pltpu.VMEM((RPW, D), jnp.float32),
            pltpu.SemaphoreType.DMA,
        ],
        compiler_params=pltpu.CompilerParams(use_tc_tiling_on_sc=False),
    )
    return f(embedding, flat_idx.reshape(ROWS // CHUNK, CHUNK))


def kernel(inputs, embedding):
    x_flat = inputs.reshape(ROWS, D)
    idx2, loss = _dist_argmin(x_flat, embedding)
    qst = _sc_gather(embedding, idx2.reshape(ROWS))
    return (qst.reshape(inputs.shape), loss.reshape(()),
            idx2.reshape(BATCH, SEQ))
